# baseline (speedup 1.0000x reference)
import functools

import jax
import jax.numpy as jnp
from jax import lax
from jax.experimental import pallas as pl
from jax.experimental.pallas import tpu as pltpu
from jax.experimental.pallas import tpu_sc as plsc

NUM_TASKS = 1000000
EMB_DIM = 64
SEQ_FEAT = 320
H0 = 256
H1 = 128
OUT_W = 85
B = 16384

_CHUNK = 128


@functools.lru_cache(maxsize=None)
def _gather_fn():
    info = plsc.get_sparse_core_info()
    nw = info.num_cores * info.num_subcores
    b_per_w = B // nw
    n_chunks = b_per_w // _CHUNK
    mesh = plsc.VectorSubcoreMesh(core_axis_name="c", subcore_axis_name="s")

    @functools.partial(
        pl.kernel,
        mesh=mesh,
        out_type=jax.ShapeDtypeStruct((B, EMB_DIM), jnp.float32),
        scratch_types=[
            pltpu.VMEM((n_chunks, _CHUNK), jnp.int32),
            pltpu.VMEM((b_per_w, EMB_DIM), jnp.float32),
            pltpu.SemaphoreType.DMA,
        ],
        compiler_params=pltpu.CompilerParams(use_tc_tiling_on_sc=False),
    )
    def gather_k(table_hbm, idx_hbm, out_hbm, idx_v, rows_v, sem):
        wid = lax.axis_index("s") * info.num_cores + lax.axis_index("c")
        pltpu.sync_copy(idx_hbm.at[wid], idx_v)
        copies = [
            pltpu.async_copy(
                table_hbm.at[idx_v.at[j]],
                rows_v.at[pl.ds(j * _CHUNK, _CHUNK)],
                sem,
            )
            for j in range(n_chunks)
        ]
        for c in copies:
            c.wait()
        pltpu.sync_copy(rows_v, out_hbm.at[pl.ds(wid * b_per_w, b_per_w)])

    return gather_k


def _mlp_body(seq_ref, emb_ref, w0s_ref, w0e_ref, s0_ref, t0_ref,
              w1_ref, s1_ref, t1_ref, wh_ref, bh_ref, out_ref):
    h = jnp.dot(seq_ref[...], w0s_ref[...], preferred_element_type=jnp.float32)
    h = h + jnp.dot(emb_ref[...], w0e_ref[...], preferred_element_type=jnp.float32)
    h = h * s0_ref[...] + t0_ref[...]
    h = jnp.maximum(h, 0.0)
    h = jnp.dot(h, w1_ref[...], preferred_element_type=jnp.float32)
    h = h * s1_ref[...] + t1_ref[...]
    h = jnp.maximum(h, 0.0)
    o = jnp.dot(h, wh_ref[...], preferred_element_type=jnp.float32) + bh_ref[...]
    col = lax.broadcasted_iota(jnp.int32, o.shape, 1)
    elu1 = jnp.where(o > 0, o, jnp.exp(jnp.minimum(o, 0.0)) - 1.0) + (1.0 + 1e-7)
    out_ref[...] = jnp.where((col >= 40) & (col < 80), elu1, o)


def kernel(sequence_input, task_input, table, W0, b0, gamma0, beta0, mm0, mv0,
           W1, b1, gamma1, beta1, mm1, mv1, Wmu, bmu, Wsig, bsig, Wpi, bpi):
    seq_flat = jnp.reshape(sequence_input, (B, SEQ_FEAT))
    idx = jnp.reshape(task_input, (32, (B // 32) // _CHUNK, _CHUNK))

    emb = _gather_fn()(table, idx)

    s0 = gamma0 / jnp.sqrt(mv0 + 1e-3)
    t0 = (b0 - mm0) * s0 + beta0
    s1 = gamma1 / jnp.sqrt(mv1 + 1e-3)
    t1 = (b1 - mm1) * s1 + beta1
    wh = jnp.concatenate([Wmu, Wsig, Wpi], axis=1)
    bh = jnp.concatenate([bmu, bsig, bpi], axis=0)

    tile = 512
    grid = (B // tile,)
    out = pl.pallas_call(
        _mlp_body,
        grid=grid,
        in_specs=[
            pl.BlockSpec((tile, SEQ_FEAT), lambda i: (i, 0)),
            pl.BlockSpec((tile, EMB_DIM), lambda i: (i, 0)),
            pl.BlockSpec((SEQ_FEAT, H0), lambda i: (0, 0)),
            pl.BlockSpec((EMB_DIM, H0), lambda i: (0, 0)),
            pl.BlockSpec((1, H0), lambda i: (0, 0)),
            pl.BlockSpec((1, H0), lambda i: (0, 0)),
            pl.BlockSpec((H0, H1), lambda i: (0, 0)),
            pl.BlockSpec((1, H1), lambda i: (0, 0)),
            pl.BlockSpec((1, H1), lambda i: (0, 0)),
            pl.BlockSpec((H1, OUT_W), lambda i: (0, 0)),
            pl.BlockSpec((1, OUT_W), lambda i: (0, 0)),
        ],
        out_specs=pl.BlockSpec((tile, OUT_W), lambda i: (i, 0)),
        out_shape=jax.ShapeDtypeStruct((B, OUT_W), jnp.float32),
    )(
        seq_flat, emb,
        W0[:SEQ_FEAT], W0[SEQ_FEAT:],
        s0[None, :], t0[None, :],
        W1, s1[None, :], t1[None, :],
        wh, bh[None, :],
    )
    return out
